# Initial kernel scaffold; baseline (speedup 1.0000x reference)
import jax
import jax.numpy as jnp
from jax.experimental import pallas as pl


def kernel(x_BD, xhat_BFD, codes_MBF, codebook, codebook_rq, W_in, b_in, W_cat, b_cat, W_r1, b_r1, W_r2, b_r2, W_out, b_out):
    raise NotImplementedError("write your pallas kernel here")


# speedup vs baseline: 1.0262x; 1.0262x over previous
import functools

import jax
import jax.numpy as jnp
from jax import lax
from jax.experimental import pallas as pl
from jax.experimental.pallas import tpu as pltpu

D = 64
DH = 64
DFF = 256
KCB = 8192
NCODES = 64
BBASE = 256
FIN = 16
FOUT = 16
B = BBASE * FIN


_A_ROWS = 16


def _dists_body(x_ref, xh_ref, rq_ref, cn_ref, out_ref):
    xt = (x_ref[...][:, None, :] - xh_ref[...]).reshape(_A_ROWS * FIN, D)
    prod = lax.dot_general(xt, rq_ref[...], (((1,), (1,)), ((), ())),
                           preferred_element_type=jnp.float32)
    out_ref[...] = cn_ref[...][None, :] - 2.0 * prod


def _stage_a(x_BD, xhat_BFD, codebook_rq, cn):
    grid = BBASE // _A_ROWS
    return pl.pallas_call(
        _dists_body,
        grid=(grid,),
        in_specs=[
            pl.BlockSpec((_A_ROWS, D), lambda i: (i, 0)),
            pl.BlockSpec((_A_ROWS, FIN, D), lambda i: (i, 0, 0)),
            pl.BlockSpec((KCB, D), lambda i: (0, 0)),
            pl.BlockSpec((KCB,), lambda i: (0,)),
        ],
        out_specs=pl.BlockSpec((_A_ROWS * FIN, KCB), lambda i: (i, 0)),
        out_shape=jax.ShapeDtypeStruct((B, KCB), jnp.float32),
    )(x_BD, xhat_BFD, codebook_rq, cn)


_C_ROWS = 8
_NC = FIN * NCODES


def _mlp_body(cw_ref, xh_ref, x_ref, tc_ref, codes_ref,
              wi_ref, bi_ref, wc_ref, bc_ref, w1_ref, b1_ref,
              w2_ref, b2_ref, wo_ref, bo_ref,
              xo_ref, co_ref):
    n = _C_ROWS * _NC
    cwf = cw_ref[...].reshape(n, D)
    xh = xh_ref[...]
    xbf = jnp.broadcast_to(
        xh.reshape(_C_ROWS * FIN, 1, D), (_C_ROWS * FIN, NCODES, D)
    ).reshape(n, D)

    h = jnp.dot(cwf, wi_ref[...], preferred_element_type=jnp.float32) + bi_ref[...][None, :]
    hc = jnp.concatenate([h, xbf], axis=-1)
    h = jnp.dot(hc, wc_ref[...], preferred_element_type=jnp.float32) + bc_ref[...][None, :]
    r = jnp.maximum(jnp.dot(h, w1_ref[...], preferred_element_type=jnp.float32)
                    + b1_ref[...][None, :], 0.0)
    h = h + (jnp.dot(r, w2_ref[...], preferred_element_type=jnp.float32)
             + b2_ref[...][None, :])
    out = jnp.dot(h, wo_ref[...], preferred_element_type=jnp.float32) + bo_ref[...][None, :]
    out = out + 1.0 * cwf
    cand = out + xbf

    cn = jnp.sum(cand * cand, axis=-1)
    xv = x_ref[...]
    cross_full = lax.dot_general(cand, xv, (((1,), (1,)), ((), ())),
                                 preferred_element_type=jnp.float32)
    cross3 = cross_full.reshape(_C_ROWS, _NC, _C_ROWS)
    bsel = (lax.broadcasted_iota(jnp.int32, (_C_ROWS, _NC, _C_ROWS), 2)
            == lax.broadcasted_iota(jnp.int32, (_C_ROWS, _NC, _C_ROWS), 0))
    cross = jnp.sum(jnp.where(bsel, cross3, 0.0), axis=-1)
    dist = cn.reshape(_C_ROWS, _NC) - 2.0 * cross

    dcur = dist
    cols = lax.broadcasted_iota(jnp.int32, (_C_ROWS, _NC), 1)
    idx_list = []
    for _ in range(FOUT):
        it = jnp.argmin(dcur, axis=-1).astype(jnp.int32)
        idx_list.append(it)
        dcur = jnp.where(cols == it[:, None], jnp.inf, dcur)
    idx = jnp.stack(idx_list, axis=-1)

    cand3 = cand.reshape(_C_ROWS, _NC, D)
    tcf = tc_ref[...].astype(jnp.float32)
    sel_rows = []
    oh_cols = lax.broadcasted_iota(jnp.int32, (FOUT, _NC), 1)
    for b in range(_C_ROWS):
        ohb = (oh_cols == idx[b][:, None]).astype(jnp.float32)
        augb = jnp.concatenate([cand3[b], tcf[b][:, None]], axis=1)
        sel_rows.append(jnp.dot(ohb, augb, preferred_element_type=jnp.float32,
                                precision=lax.Precision.HIGHEST))
    sel = jnp.stack(sel_rows, axis=0)
    xo_ref[...] = sel[:, :, :D]
    codes_g = (sel[:, :, D] + 0.5).astype(jnp.int32)
    fidx = lax.shift_right_logical(idx, 6)
    fidxb = jnp.broadcast_to(fidx[None], (4, _C_ROWS, FOUT))
    csel = jnp.take_along_axis(codes_ref[...], fidxb, axis=-1)
    co_ref[...] = jnp.concatenate([csel, codes_g[None]], axis=0)


def _stage_c(cw, xhat_BFD, x_BD, tc_flat, codes_MBF,
             W_in, b_in, W_cat, b_cat, W_r1, b_r1, W_r2, b_r2, W_out, b_out):
    grid = BBASE // _C_ROWS
    const = lambda *dims: pl.BlockSpec(dims, lambda i: (0,) * len(dims))
    return pl.pallas_call(
        _mlp_body,
        grid=(grid,),
        in_specs=[
            pl.BlockSpec((_C_ROWS, _NC, D), lambda i: (i, 0, 0)),
            pl.BlockSpec((_C_ROWS, FIN, D), lambda i: (i, 0, 0)),
            pl.BlockSpec((_C_ROWS, D), lambda i: (i, 0)),
            pl.BlockSpec((_C_ROWS, _NC), lambda i: (i, 0)),
            pl.BlockSpec((4, _C_ROWS, FOUT), lambda i: (0, i, 0)),
            const(D, DH), const(DH), const(DH + D, DH), const(DH),
            const(DH, DFF), const(DFF), const(DFF, DH), const(DH),
            const(DH, D), const(D),
        ],
        out_specs=[
            pl.BlockSpec((_C_ROWS, FOUT, D), lambda i: (i, 0, 0)),
            pl.BlockSpec((5, _C_ROWS, FOUT), lambda i: (0, i, 0)),
        ],
        out_shape=[
            jax.ShapeDtypeStruct((BBASE, FOUT, D), jnp.float32),
            jax.ShapeDtypeStruct((5, BBASE, FOUT), jnp.int32),
        ],
    )(cw, xhat_BFD, x_BD, tc_flat, codes_MBF,
      W_in, b_in, W_cat, b_cat, W_r1, b_r1, W_r2, b_r2, W_out, b_out)


def kernel(x_BD, xhat_BFD, codes_MBF, codebook, codebook_rq,
           W_in, b_in, W_cat, b_cat, W_r1, b_r1, W_r2, b_r2, W_out, b_out):
    cn = jnp.sum(codebook_rq * codebook_rq, axis=-1)
    dists = _stage_a(x_BD, xhat_BFD, codebook_rq, cn)
    _, tc64 = lax.top_k(-dists, NCODES)
    cw = jnp.take(codebook, tc64, axis=0)
    tc_flat = tc64.reshape(BBASE, _NC)
    cw3 = cw.reshape(BBASE, _NC, D)
    xhat_next, codes_out = _stage_c(
        cw3, xhat_BFD, x_BD, tc_flat, codes_MBF,
        W_in, b_in, W_cat, b_cat, W_r1, b_r1, W_r2, b_r2, W_out, b_out)
    return (xhat_next, codes_out)


# speedup vs baseline: 8.4050x; 8.1906x over previous
import functools

import jax
import jax.numpy as jnp
from jax import lax
from jax.experimental import pallas as pl
from jax.experimental.pallas import tpu as pltpu
from jax.experimental.pallas import tpu_sc as plsc

D = 64
DH = 64
DFF = 256
KCB = 8192
NCODES = 64
BBASE = 256
FIN = 16
FOUT = 16
B = BBASE * FIN


_A_ROWS = 16


def _dists_body(x_ref, xh_ref, rq_ref, cn_ref, out_ref):
    xt = (x_ref[...][:, None, :] - xh_ref[...]).reshape(_A_ROWS * FIN, D)
    prod = lax.dot_general(xt, rq_ref[...], (((1,), (1,)), ((), ())),
                           preferred_element_type=jnp.float32)
    out_ref[...] = cn_ref[...][None, :] - 2.0 * prod


def _stage_a(x_BD, xhat_BFD, codebook_rq, cn):
    grid = BBASE // _A_ROWS
    return pl.pallas_call(
        _dists_body,
        grid=(grid,),
        in_specs=[
            pl.BlockSpec((_A_ROWS, D), lambda i: (i, 0)),
            pl.BlockSpec((_A_ROWS, FIN, D), lambda i: (i, 0, 0)),
            pl.BlockSpec((KCB, D), lambda i: (0, 0)),
            pl.BlockSpec((KCB,), lambda i: (0,)),
        ],
        out_specs=pl.BlockSpec((_A_ROWS * FIN, KCB), lambda i: (i, 0)),
        out_shape=jax.ShapeDtypeStruct((B, KCB), jnp.float32),
    )(x_BD, xhat_BFD, codebook_rq, cn)


_SC_CORES = 2
_SC_SUBCORES = 16
_SC_WORKERS = _SC_CORES * _SC_SUBCORES
_ROWS_PER_W = B // _SC_WORKERS


def _key_s(v):
    ks = plsc.bitcast(v, jnp.int32)
    m = lax.shift_right_arithmetic(ks, 31)
    return lax.bitwise_xor(ks, lax.bitwise_and(m, jnp.int32(0x7FFFFFFF)))


def _digit(ks, shift, first):
    b = lax.bitwise_and(lax.shift_right_logical(ks, shift), jnp.int32(255))
    if first:
        b = lax.bitwise_xor(b, jnp.int32(128))
    return b


def _popcnt(m):
    return plsc.all_reduce_population_count(m)


def _sc_body(dists_hbm, cb_hbm, tc_hbm, cw_hbm,
             d0, d1, hist, ck0, ci0, ck1, ci1, oidx, gidx, grow, growc,
             sem0, sem1, gsem):
    lane = lax.iota(jnp.int32, 16)
    ones16 = jnp.ones((16,), jnp.int32)
    zeros16 = jnp.zeros((16,), jnp.int32)
    wid = lax.axis_index("s") * _SC_CORES + lax.axis_index("c")
    base = wid * _ROWS_PER_W

    def zero_hist(i, _):
        hist[pl.ds(16 * i, 16)] = zeros16
        return 0

    def scan_hist(k):
        def s(i, carry):
            tot, bstar, cless = carry
            hv = hist[pl.ds(16 * i, 16)]
            cum = plsc.cumsum(hv) + tot
            m = cum >= k
            ffs = jnp.max(plsc.all_reduce_ffs(m))
            cl_new = jnp.maximum(tot, jnp.max(jnp.where(m, 0, cum)))
            take = jnp.logical_and(bstar < 0, ffs < 16)
            bstar = jnp.where(take, 16 * i + ffs, bstar)
            cless = jnp.where(take, cl_new, cless)
            tot = jnp.max(cum)
            return tot, bstar, cless
        _, bstar, cless = lax.fori_loop(0, 16, s, (jnp.int32(0), jnp.int32(-1), jnp.int32(0)))
        return bstar, cless

    def refine_level(shift, first, last, srck, srci, dstk, dsti,
                     offw, k_rem, ncand):
        lax.fori_loop(0, 16, zero_hist, 0)
        niter = (ncand + 15) // 16

        def h(j, _):
            ks = srck[pl.ds(16 * j, 16)]
            valid = (lane + 16 * j) < ncand
            plsc.addupdate_scatter(hist, [_digit(ks, shift, first)], ones16,
                                   mask=valid)
            return 0
        lax.fori_loop(0, niter, h, 0)
        bstar, cless = scan_hist(k_rem)
        krem2 = zeros16 + (k_rem - cless)

        def e(j, carry):
            offw, offc, krem2 = carry
            ks = srck[pl.ds(16 * j, 16)]
            iv = srci[pl.ds(16 * j, 16)]
            valid = (lane + 16 * j) < ncand
            d = _digit(ks, shift, first)
            mw = jnp.logical_and(valid, d < bstar)
            mc = jnp.logical_and(valid, d == bstar)
            if last:
                trank = plsc.cumsum(jnp.where(mc, 1, 0))
                mtake = jnp.logical_and(mc, trank <= krem2)
                krem2 = krem2 - _popcnt(mtake)
                mw = jnp.logical_or(mw, mtake)
            posw = offw + plsc.cumsum(jnp.where(mw, 1, 0)) - 1
            plsc.store_scatter(oidx, [posw], iv, mask=mw)
            offw = offw + _popcnt(mw)
            if not last:
                posc = offc + plsc.cumsum(jnp.where(mc, 1, 0)) - 1
                plsc.store_scatter(dstk, [posc], ks, mask=mc)
                plsc.store_scatter(dsti, [posc], iv, mask=mc)
                offc = offc + _popcnt(mc)
            return offw, offc, krem2
        offw, offc, _ = lax.fori_loop(0, niter, e, (offw, zeros16, krem2))
        return offw, k_rem - cless, jnp.max(offc)

    def select_row(drow):
        def ga(i, acc):
            a0, a1, a2, a3 = acc
            a0 = jnp.minimum(a0, drow[pl.ds(64 * i, 16)])
            a1 = jnp.minimum(a1, drow[pl.ds(64 * i + 16, 16)])
            a2 = jnp.minimum(a2, drow[pl.ds(64 * i + 32, 16)])
            a3 = jnp.minimum(a3, drow[pl.ds(64 * i + 48, 16)])
            return a0, a1, a2, a3
        inf16 = jnp.full((16,), jnp.inf, jnp.float32)
        a0, a1, a2, a3 = lax.fori_loop(0, 128, ga, (inf16, inf16, inf16, inf16))
        t = jnp.max(jnp.maximum(jnp.maximum(a0, a1), jnp.maximum(a2, a3)))

        def cb(j, offc):
            v = drow[pl.ds(16 * j, 16)]
            m = v <= t
            ks = _key_s(v)
            iv = lane + 16 * j
            posc = offc + plsc.cumsum(jnp.where(m, 1, 0)) - 1
            plsc.store_scatter(ck0, [posc], ks, mask=m)
            plsc.store_scatter(ci0, [posc], iv, mask=m)
            return offc + _popcnt(m)
        offc = lax.fori_loop(0, KCB // 16, cb, zeros16)
        ncand = jnp.max(offc)

        offw = zeros16
        offw, k_rem, ncand = refine_level(24, True, False, ck0, ci0, ck1, ci1,
                                          offw, jnp.int32(NCODES), ncand)
        offw, k_rem, ncand = refine_level(16, False, False, ck1, ci1, ck0, ci0,
                                          offw, k_rem, ncand)
        offw, k_rem, ncand = refine_level(8, False, False, ck0, ci0, ck1, ci1,
                                          offw, k_rem, ncand)
        refine_level(0, False, True, ck1, ci1, ck0, ci0, offw, k_rem, ncand)

    def handle_row(r, drow, dsem):
        pltpu.make_async_copy(dists_hbm.at[base + r], drow, dsem).wait()
        select_row(drow)

        @pl.when(r + 2 < _ROWS_PER_W)
        def _():
            pltpu.make_async_copy(dists_hbm.at[base + r + 2], drow, dsem).start()

        for q in range(4):
            gidx[pl.ds(16 * q, 16)] = oidx[pl.ds(16 * q, 16)]
        pltpu.async_copy(cb_hbm.at[gidx], grow, gsem).wait()
        for q in range(NCODES):
            for h in range(4):
                growc[q, pl.ds(16 * h, 16)] = grow[q, pl.ds(16 * h, 16)]
        pltpu.sync_copy(growc, cw_hbm.at[base + r])
        pltpu.sync_copy(gidx, tc_hbm.at[base + r])

    pltpu.make_async_copy(dists_hbm.at[base], d0, sem0).start()
    pltpu.make_async_copy(dists_hbm.at[base + 1], d1, sem1).start()

    def row_pair(i, _):
        handle_row(2 * i, d0, sem0)
        handle_row(2 * i + 1, d1, sem1)
        return 0
    lax.fori_loop(0, _ROWS_PER_W // 2, row_pair, 0)


def _sc_topk_gather(dists, codebook):
    mesh = plsc.VectorSubcoreMesh(core_axis_name="c", subcore_axis_name="s",
                                  num_cores=_SC_CORES,
                                  num_subcores=_SC_SUBCORES)
    f = pl.kernel(
        _sc_body,
        out_type=[
            jax.ShapeDtypeStruct((B, NCODES), jnp.int32),
            jax.ShapeDtypeStruct((B, NCODES, D), jnp.float32),
        ],
        mesh=mesh,
        compiler_params=pltpu.CompilerParams(needs_layout_passes=False),
        scratch_types=[
            pltpu.VMEM((KCB,), jnp.float32),
            pltpu.VMEM((KCB,), jnp.float32),
            pltpu.VMEM((256,), jnp.int32),
            pltpu.VMEM((KCB + 16,), jnp.int32),
            pltpu.VMEM((KCB + 16,), jnp.int32),
            pltpu.VMEM((KCB + 16,), jnp.int32),
            pltpu.VMEM((KCB + 16,), jnp.int32),
            pltpu.VMEM((NCODES + 16,), jnp.int32),
            pltpu.VMEM((NCODES,), jnp.int32),
            pltpu.VMEM((NCODES, 128), jnp.float32),
            pltpu.VMEM((NCODES, D), jnp.float32),
            pltpu.SemaphoreType.DMA,
            pltpu.SemaphoreType.DMA,
            pltpu.SemaphoreType.DMA,
        ],
    )
    cb_pad = jnp.pad(codebook, ((0, 0), (0, 128 - D)))
    return f(dists, cb_pad)


_C_ROWS = 8
_NC = FIN * NCODES


def _mlp_body(cw_ref, xh_ref, x_ref, tc_ref, codes_ref,
              wi_ref, bi_ref, wc_ref, bc_ref, w1_ref, b1_ref,
              w2_ref, b2_ref, wo_ref, bo_ref,
              xo_ref, co_ref):
    n = _C_ROWS * _NC
    cwf = cw_ref[...].reshape(n, D)
    xh = xh_ref[...]
    xbf = jnp.broadcast_to(
        xh.reshape(_C_ROWS * FIN, 1, D), (_C_ROWS * FIN, NCODES, D)
    ).reshape(n, D)

    h = jnp.dot(cwf, wi_ref[...], preferred_element_type=jnp.float32) + bi_ref[...][None, :]
    hc = jnp.concatenate([h, xbf], axis=-1)
    h = jnp.dot(hc, wc_ref[...], preferred_element_type=jnp.float32) + bc_ref[...][None, :]
    r = jnp.maximum(jnp.dot(h, w1_ref[...], preferred_element_type=jnp.float32)
                    + b1_ref[...][None, :], 0.0)
    h = h + (jnp.dot(r, w2_ref[...], preferred_element_type=jnp.float32)
             + b2_ref[...][None, :])
    out = jnp.dot(h, wo_ref[...], preferred_element_type=jnp.float32) + bo_ref[...][None, :]
    out = out + 1.0 * cwf
    cand = out + xbf

    cn = jnp.sum(cand * cand, axis=-1)
    xv = x_ref[...]
    cross_full = lax.dot_general(cand, xv, (((1,), (1,)), ((), ())),
                                 preferred_element_type=jnp.float32)
    cross3 = cross_full.reshape(_C_ROWS, _NC, _C_ROWS)
    bsel = (lax.broadcasted_iota(jnp.int32, (_C_ROWS, _NC, _C_ROWS), 2)
            == lax.broadcasted_iota(jnp.int32, (_C_ROWS, _NC, _C_ROWS), 0))
    cross = jnp.sum(jnp.where(bsel, cross3, 0.0), axis=-1)
    dist = cn.reshape(_C_ROWS, _NC) - 2.0 * cross

    dcur = dist
    cols = lax.broadcasted_iota(jnp.int32, (_C_ROWS, _NC), 1)
    idx_list = []
    for _ in range(FOUT):
        it = jnp.argmin(dcur, axis=-1).astype(jnp.int32)
        idx_list.append(it)
        dcur = jnp.where(cols == it[:, None], jnp.inf, dcur)
    idx = jnp.stack(idx_list, axis=-1)

    cand3 = cand.reshape(_C_ROWS, _NC, D)
    tcf = tc_ref[...].astype(jnp.float32)
    sel_rows = []
    oh_cols = lax.broadcasted_iota(jnp.int32, (FOUT, _NC), 1)
    for b in range(_C_ROWS):
        ohb = (oh_cols == idx[b][:, None]).astype(jnp.float32)
        augb = jnp.concatenate([cand3[b], tcf[b][:, None]], axis=1)
        sel_rows.append(jnp.dot(ohb, augb, preferred_element_type=jnp.float32,
                                precision=lax.Precision.HIGHEST))
    sel = jnp.stack(sel_rows, axis=0)
    xo_ref[...] = sel[:, :, :D]
    codes_g = (sel[:, :, D] + 0.5).astype(jnp.int32)
    fidx = lax.shift_right_logical(idx, 6)
    fidxb = jnp.broadcast_to(fidx[None], (4, _C_ROWS, FOUT))
    csel = jnp.take_along_axis(codes_ref[...], fidxb, axis=-1)
    co_ref[...] = jnp.concatenate([csel, codes_g[None]], axis=0)


def _stage_c(cw, xhat_BFD, x_BD, tc_flat, codes_MBF,
             W_in, b_in, W_cat, b_cat, W_r1, b_r1, W_r2, b_r2, W_out, b_out):
    grid = BBASE // _C_ROWS
    const = lambda *dims: pl.BlockSpec(dims, lambda i: (0,) * len(dims))
    return pl.pallas_call(
        _mlp_body,
        grid=(grid,),
        in_specs=[
            pl.BlockSpec((_C_ROWS, _NC, D), lambda i: (i, 0, 0)),
            pl.BlockSpec((_C_ROWS, FIN, D), lambda i: (i, 0, 0)),
            pl.BlockSpec((_C_ROWS, D), lambda i: (i, 0)),
            pl.BlockSpec((_C_ROWS, _NC), lambda i: (i, 0)),
            pl.BlockSpec((4, _C_ROWS, FOUT), lambda i: (0, i, 0)),
            const(D, DH), const(DH), const(DH + D, DH), const(DH),
            const(DH, DFF), const(DFF), const(DFF, DH), const(DH),
            const(DH, D), const(D),
        ],
        out_specs=[
            pl.BlockSpec((_C_ROWS, FOUT, D), lambda i: (i, 0, 0)),
            pl.BlockSpec((5, _C_ROWS, FOUT), lambda i: (0, i, 0)),
        ],
        out_shape=[
            jax.ShapeDtypeStruct((BBASE, FOUT, D), jnp.float32),
            jax.ShapeDtypeStruct((5, BBASE, FOUT), jnp.int32),
        ],
    )(cw, xhat_BFD, x_BD, tc_flat, codes_MBF,
      W_in, b_in, W_cat, b_cat, W_r1, b_r1, W_r2, b_r2, W_out, b_out)


def kernel(x_BD, xhat_BFD, codes_MBF, codebook, codebook_rq,
           W_in, b_in, W_cat, b_cat, W_r1, b_r1, W_r2, b_r2, W_out, b_out):
    cn = jnp.sum(codebook_rq * codebook_rq, axis=-1)
    dists = _stage_a(x_BD, xhat_BFD, codebook_rq, cn)
    tc64, cw = _sc_topk_gather(dists, codebook)
    tc_flat = tc64.reshape(BBASE, _NC)
    cw3 = cw.reshape(BBASE, _NC, D)
    xhat_next, codes_out = _stage_c(
        cw3, xhat_BFD, x_BD, tc_flat, codes_MBF,
        W_in, b_in, W_cat, b_cat, W_r1, b_r1, W_r2, b_r2, W_out, b_out)
    return (xhat_next, codes_out)


# speedup vs baseline: 9.0146x; 1.0725x over previous
import functools

import jax
import jax.numpy as jnp
from jax import lax
from jax.experimental import pallas as pl
from jax.experimental.pallas import tpu as pltpu
from jax.experimental.pallas import tpu_sc as plsc

D = 64
DH = 64
DFF = 256
KCB = 8192
NCODES = 64
BBASE = 256
FIN = 16
FOUT = 16
B = BBASE * FIN


_A_ROWS = 16


def _dists_body(x_ref, xh_ref, rq_ref, cn_ref, out_ref):
    xt = (x_ref[...][:, None, :] - xh_ref[...]).reshape(_A_ROWS * FIN, D)
    prod = lax.dot_general(xt, rq_ref[...], (((1,), (1,)), ((), ())),
                           preferred_element_type=jnp.float32)
    out_ref[...] = cn_ref[...][None, :] - 2.0 * prod


def _stage_a(x_BD, xhat_BFD, codebook_rq, cn):
    grid = BBASE // _A_ROWS
    return pl.pallas_call(
        _dists_body,
        grid=(grid,),
        in_specs=[
            pl.BlockSpec((_A_ROWS, D), lambda i: (i, 0)),
            pl.BlockSpec((_A_ROWS, FIN, D), lambda i: (i, 0, 0)),
            pl.BlockSpec((KCB, D), lambda i: (0, 0)),
            pl.BlockSpec((KCB,), lambda i: (0,)),
        ],
        out_specs=pl.BlockSpec((_A_ROWS * FIN, KCB), lambda i: (i, 0)),
        out_shape=jax.ShapeDtypeStruct((B, KCB), jnp.float32),
    )(x_BD, xhat_BFD, codebook_rq, cn)


_SC_CORES = 2
_SC_SUBCORES = 16
_SC_WORKERS = _SC_CORES * _SC_SUBCORES
_ROWS_PER_W = B // _SC_WORKERS


def _key_s(v):
    ks = plsc.bitcast(v, jnp.int32)
    m = lax.shift_right_arithmetic(ks, 31)
    return lax.bitwise_xor(ks, lax.bitwise_and(m, jnp.int32(0x7FFFFFFF)))


def _digit(ks, shift, first):
    b = lax.bitwise_and(lax.shift_right_logical(ks, shift), jnp.int32(15))
    if first:
        b = lax.bitwise_xor(b, jnp.int32(8))
    return b


def _popcnt(m):
    return plsc.all_reduce_population_count(m)


def _sc_body(dists_hbm, cb_hbm, tc_hbm, cw_hbm,
             d0, d1, hist, ck0, ci0, ck1, ci1, oidx, gidx, gidx2,
             grow, grow2, growc, sem0, sem1, gsem, gsem2):
    lane = lax.iota(jnp.int32, 16)
    ones16 = jnp.ones((16,), jnp.int32)
    zeros16 = jnp.zeros((16,), jnp.int32)
    wid = lax.axis_index("s") * _SC_CORES + lax.axis_index("c")
    base = wid * _ROWS_PER_W

    def scan_hist(k):
        hv = hist[pl.ds(0, 16)]
        cum = plsc.cumsum(hv)
        m = cum >= k
        bstar = jnp.max(plsc.all_reduce_ffs(m))
        cless = jnp.max(jnp.where(m, 0, cum))
        return bstar, cless

    def refine_level(shift, first, last, srck, srci, dstk, dsti,
                     offw, k_rem, ncand):
        hist[pl.ds(0, 16)] = zeros16
        niter = (ncand + 15) // 16

        def h(j, _):
            ks = srck[pl.ds(16 * j, 16)]
            valid = (lane + 16 * j) < ncand
            plsc.addupdate_scatter(hist, [_digit(ks, shift, first)], ones16,
                                   mask=valid)
            return 0
        lax.fori_loop(0, niter, h, 0)
        bstar, cless = scan_hist(k_rem)
        krem2 = zeros16 + (k_rem - cless)

        def e(j, carry):
            offw, offc, krem2 = carry
            ks = srck[pl.ds(16 * j, 16)]
            iv = srci[pl.ds(16 * j, 16)]
            valid = (lane + 16 * j) < ncand
            d = _digit(ks, shift, first)
            mw = jnp.logical_and(valid, d < bstar)
            mc = jnp.logical_and(valid, d == bstar)
            if last:
                trank = plsc.cumsum(jnp.where(mc, 1, 0))
                mtake = jnp.logical_and(mc, trank <= krem2)
                krem2 = krem2 - _popcnt(mtake)
                mw = jnp.logical_or(mw, mtake)
            posw = offw + plsc.cumsum(jnp.where(mw, 1, 0)) - 1
            plsc.store_scatter(oidx, [posw], iv, mask=mw)
            offw = offw + _popcnt(mw)
            if not last:
                posc = offc + plsc.cumsum(jnp.where(mc, 1, 0)) - 1
                plsc.store_scatter(dstk, [posc], ks, mask=mc)
                plsc.store_scatter(dsti, [posc], iv, mask=mc)
                offc = offc + _popcnt(mc)
            return offw, offc, krem2
        offw, offc, _ = lax.fori_loop(0, niter, e, (offw, zeros16, krem2))
        return offw, k_rem - cless, jnp.max(offc)

    def select_row(drow):
        def ga(i, acc):
            a0, a1, a2, a3 = acc
            a0 = jnp.minimum(a0, drow[pl.ds(64 * i, 16)])
            a1 = jnp.minimum(a1, drow[pl.ds(64 * i + 16, 16)])
            a2 = jnp.minimum(a2, drow[pl.ds(64 * i + 32, 16)])
            a3 = jnp.minimum(a3, drow[pl.ds(64 * i + 48, 16)])
            return a0, a1, a2, a3
        inf16 = jnp.full((16,), jnp.inf, jnp.float32)
        a0, a1, a2, a3 = lax.fori_loop(0, 128, ga, (inf16, inf16, inf16, inf16))
        t = jnp.max(jnp.maximum(jnp.maximum(a0, a1), jnp.maximum(a2, a3)))

        def cb(j, offc):
            for u in range(4):
                v = drow[pl.ds(64 * j + 16 * u, 16)]
                m = v <= t
                ks = _key_s(v)
                iv = lane + (64 * j + 16 * u)
                posc = offc + plsc.cumsum(jnp.where(m, 1, 0)) - 1
                plsc.store_scatter(ck0, [posc], ks, mask=m)
                plsc.store_scatter(ci0, [posc], iv, mask=m)
                offc = offc + _popcnt(m)
            return offc
        offc = lax.fori_loop(0, KCB // 64, cb, zeros16)
        ncand = jnp.max(offc)

        offw = zeros16
        k_rem = jnp.int32(NCODES)
        bufs = [(ck0, ci0, ck1, ci1), (ck1, ci1, ck0, ci0)]
        for lvl in range(8):
            sk, si, dk, di = bufs[lvl % 2]
            if lvl < 7:
                offw, k_rem, ncand = refine_level(28 - 4 * lvl, lvl == 0, False,
                                                  sk, si, dk, di,
                                                  offw, k_rem, ncand)
            else:
                refine_level(0, False, True, sk, si, dk, di, offw, k_rem, ncand)

    gidxs = (gidx, gidx2)
    grows = (grow, grow2)
    gsems = (gsem, gsem2)

    def writeback(r, p):
        pltpu.make_async_copy(cb_hbm.at[gidxs[p]], grows[p], gsems[p]).wait()
        for q in range(NCODES):
            for h in range(4):
                growc[q, pl.ds(16 * h, 16)] = grows[p][q, pl.ds(16 * h, 16)]
        pltpu.sync_copy(growc, cw_hbm.at[base + r])
        pltpu.sync_copy(gidxs[p], tc_hbm.at[base + r])

    def handle_row(r, drow, dsem, p):
        pltpu.make_async_copy(dists_hbm.at[base + r], drow, dsem).wait()
        select_row(drow)
        for q in range(4):
            gidxs[p][pl.ds(16 * q, 16)] = oidx[pl.ds(16 * q, 16)]
        pltpu.make_async_copy(cb_hbm.at[gidxs[p]], grows[p], gsems[p]).start()

        @pl.when(r + 2 < _ROWS_PER_W)
        def _():
            pltpu.make_async_copy(dists_hbm.at[base + r + 2], drow, dsem).start()

        @pl.when(r >= 1)
        def _():
            writeback(r - 1, 1 - p)

    pltpu.make_async_copy(dists_hbm.at[base], d0, sem0).start()
    pltpu.make_async_copy(dists_hbm.at[base + 1], d1, sem1).start()

    def row_pair(i, _):
        handle_row(2 * i, d0, sem0, 0)
        handle_row(2 * i + 1, d1, sem1, 1)
        return 0
    lax.fori_loop(0, _ROWS_PER_W // 2, row_pair, 0)
    writeback(jnp.int32(_ROWS_PER_W - 1), 1)


def _sc_topk_gather(dists, codebook):
    mesh = plsc.VectorSubcoreMesh(core_axis_name="c", subcore_axis_name="s",
                                  num_cores=_SC_CORES,
                                  num_subcores=_SC_SUBCORES)
    f = pl.kernel(
        _sc_body,
        out_type=[
            jax.ShapeDtypeStruct((B, NCODES), jnp.int32),
            jax.ShapeDtypeStruct((B, NCODES, D), jnp.float32),
        ],
        mesh=mesh,
        compiler_params=pltpu.CompilerParams(needs_layout_passes=False),
        scratch_types=[
            pltpu.VMEM((KCB,), jnp.float32),
            pltpu.VMEM((KCB,), jnp.float32),
            pltpu.VMEM((256,), jnp.int32),
            pltpu.VMEM((KCB + 16,), jnp.int32),
            pltpu.VMEM((KCB + 16,), jnp.int32),
            pltpu.VMEM((KCB + 16,), jnp.int32),
            pltpu.VMEM((KCB + 16,), jnp.int32),
            pltpu.VMEM((NCODES + 16,), jnp.int32),
            pltpu.VMEM((NCODES,), jnp.int32),
            pltpu.VMEM((NCODES,), jnp.int32),
            pltpu.VMEM((NCODES, 128), jnp.float32),
            pltpu.VMEM((NCODES, 128), jnp.float32),
            pltpu.VMEM((NCODES, D), jnp.float32),
            pltpu.SemaphoreType.DMA,
            pltpu.SemaphoreType.DMA,
            pltpu.SemaphoreType.DMA,
            pltpu.SemaphoreType.DMA,
        ],
    )
    cb_pad = jnp.pad(codebook, ((0, 0), (0, 128 - D)))
    return f(dists, cb_pad)


_C_ROWS = 8
_NC = FIN * NCODES


def _mlp_body(cw_ref, xh_ref, x_ref, tc_ref, codes_ref,
              wi_ref, bi_ref, wc_ref, bc_ref, w1_ref, b1_ref,
              w2_ref, b2_ref, wo_ref, bo_ref,
              xo_ref, co_ref):
    n = _C_ROWS * _NC
    cwf = cw_ref[...].reshape(n, D)
    xh = xh_ref[...]
    xbf = jnp.broadcast_to(
        xh.reshape(_C_ROWS * FIN, 1, D), (_C_ROWS * FIN, NCODES, D)
    ).reshape(n, D)

    h = jnp.dot(cwf, wi_ref[...], preferred_element_type=jnp.float32) + bi_ref[...][None, :]
    hc = jnp.concatenate([h, xbf], axis=-1)
    h = jnp.dot(hc, wc_ref[...], preferred_element_type=jnp.float32) + bc_ref[...][None, :]
    r = jnp.maximum(jnp.dot(h, w1_ref[...], preferred_element_type=jnp.float32)
                    + b1_ref[...][None, :], 0.0)
    h = h + (jnp.dot(r, w2_ref[...], preferred_element_type=jnp.float32)
             + b2_ref[...][None, :])
    out = jnp.dot(h, wo_ref[...], preferred_element_type=jnp.float32) + bo_ref[...][None, :]
    out = out + 1.0 * cwf
    cand = out + xbf

    cn = jnp.sum(cand * cand, axis=-1)
    xv = x_ref[...]
    cross_full = lax.dot_general(cand, xv, (((1,), (1,)), ((), ())),
                                 preferred_element_type=jnp.float32)
    cross3 = cross_full.reshape(_C_ROWS, _NC, _C_ROWS)
    bsel = (lax.broadcasted_iota(jnp.int32, (_C_ROWS, _NC, _C_ROWS), 2)
            == lax.broadcasted_iota(jnp.int32, (_C_ROWS, _NC, _C_ROWS), 0))
    cross = jnp.sum(jnp.where(bsel, cross3, 0.0), axis=-1)
    dist = cn.reshape(_C_ROWS, _NC) - 2.0 * cross

    dcur = dist
    cols = lax.broadcasted_iota(jnp.int32, (_C_ROWS, _NC), 1)
    idx_list = []
    for _ in range(FOUT):
        it = jnp.argmin(dcur, axis=-1).astype(jnp.int32)
        idx_list.append(it)
        dcur = jnp.where(cols == it[:, None], jnp.inf, dcur)
    idx = jnp.stack(idx_list, axis=-1)

    cand3 = cand.reshape(_C_ROWS, _NC, D)
    tcf = tc_ref[...].astype(jnp.float32)
    sel_rows = []
    oh_cols = lax.broadcasted_iota(jnp.int32, (FOUT, _NC), 1)
    for b in range(_C_ROWS):
        ohb = (oh_cols == idx[b][:, None]).astype(jnp.float32)
        augb = jnp.concatenate([cand3[b], tcf[b][:, None]], axis=1)
        sel_rows.append(jnp.dot(ohb, augb, preferred_element_type=jnp.float32,
                                precision=lax.Precision.HIGHEST))
    sel = jnp.stack(sel_rows, axis=0)
    xo_ref[...] = sel[:, :, :D]
    codes_g = (sel[:, :, D] + 0.5).astype(jnp.int32)
    fidx = lax.shift_right_logical(idx, 6)
    fidxb = jnp.broadcast_to(fidx[None], (4, _C_ROWS, FOUT))
    csel = jnp.take_along_axis(codes_ref[...], fidxb, axis=-1)
    co_ref[...] = jnp.concatenate([csel, codes_g[None]], axis=0)


def _stage_c(cw, xhat_BFD, x_BD, tc_flat, codes_MBF,
             W_in, b_in, W_cat, b_cat, W_r1, b_r1, W_r2, b_r2, W_out, b_out):
    grid = BBASE // _C_ROWS
    const = lambda *dims: pl.BlockSpec(dims, lambda i: (0,) * len(dims))
    return pl.pallas_call(
        _mlp_body,
        grid=(grid,),
        in_specs=[
            pl.BlockSpec((_C_ROWS, _NC, D), lambda i: (i, 0, 0)),
            pl.BlockSpec((_C_ROWS, FIN, D), lambda i: (i, 0, 0)),
            pl.BlockSpec((_C_ROWS, D), lambda i: (i, 0)),
            pl.BlockSpec((_C_ROWS, _NC), lambda i: (i, 0)),
            pl.BlockSpec((4, _C_ROWS, FOUT), lambda i: (0, i, 0)),
            const(D, DH), const(DH), const(DH + D, DH), const(DH),
            const(DH, DFF), const(DFF), const(DFF, DH), const(DH),
            const(DH, D), const(D),
        ],
        out_specs=[
            pl.BlockSpec((_C_ROWS, FOUT, D), lambda i: (i, 0, 0)),
            pl.BlockSpec((5, _C_ROWS, FOUT), lambda i: (0, i, 0)),
        ],
        out_shape=[
            jax.ShapeDtypeStruct((BBASE, FOUT, D), jnp.float32),
            jax.ShapeDtypeStruct((5, BBASE, FOUT), jnp.int32),
        ],
    )(cw, xhat_BFD, x_BD, tc_flat, codes_MBF,
      W_in, b_in, W_cat, b_cat, W_r1, b_r1, W_r2, b_r2, W_out, b_out)


def kernel(x_BD, xhat_BFD, codes_MBF, codebook, codebook_rq,
           W_in, b_in, W_cat, b_cat, W_r1, b_r1, W_r2, b_r2, W_out, b_out):
    cn = jnp.sum(codebook_rq * codebook_rq, axis=-1)
    dists = _stage_a(x_BD, xhat_BFD, codebook_rq, cn)
    tc64, cw = _sc_topk_gather(dists, codebook)
    tc_flat = tc64.reshape(BBASE, _NC)
    cw3 = cw.reshape(BBASE, _NC, D)
    xhat_next, codes_out = _stage_c(
        cw3, xhat_BFD, x_BD, tc_flat, codes_MBF,
        W_in, b_in, W_cat, b_cat, W_r1, b_r1, W_r2, b_r2, W_out, b_out)
    return (xhat_next, codes_out)


# speedup vs baseline: 14.5627x; 1.6155x over previous
import functools

import jax
import jax.numpy as jnp
from jax import lax
from jax.experimental import pallas as pl
from jax.experimental.pallas import tpu as pltpu
from jax.experimental.pallas import tpu_sc as plsc

D = 64
DH = 64
DFF = 256
KCB = 8192
NCODES = 64
BBASE = 256
FIN = 16
FOUT = 16
B = BBASE * FIN


_A_ROWS = 16


def _dists_body(x_ref, xh_ref, rq_ref, cn_ref, out_ref):
    xt = (x_ref[...][:, None, :] - xh_ref[...]).reshape(_A_ROWS * FIN, D)
    prod = lax.dot_general(xt, rq_ref[...], (((1,), (1,)), ((), ())),
                           preferred_element_type=jnp.float32)
    out_ref[...] = cn_ref[...][None, :] - 2.0 * prod


def _stage_a(x_BD, xhat_BFD, codebook_rq, cn):
    grid = BBASE // _A_ROWS
    return pl.pallas_call(
        _dists_body,
        grid=(grid,),
        in_specs=[
            pl.BlockSpec((_A_ROWS, D), lambda i: (i, 0)),
            pl.BlockSpec((_A_ROWS, FIN, D), lambda i: (i, 0, 0)),
            pl.BlockSpec((KCB, D), lambda i: (0, 0)),
            pl.BlockSpec((KCB,), lambda i: (0,)),
        ],
        out_specs=pl.BlockSpec((_A_ROWS * FIN, KCB), lambda i: (i, 0)),
        out_shape=jax.ShapeDtypeStruct((B, KCB), jnp.float32),
    )(x_BD, xhat_BFD, codebook_rq, cn)


_SC_CORES = 2
_SC_SUBCORES = 16
_SC_WORKERS = _SC_CORES * _SC_SUBCORES
_ROWS_PER_W = B // _SC_WORKERS


def _key_s(v):
    ks = plsc.bitcast(v, jnp.int32)
    m = lax.shift_right_arithmetic(ks, 31)
    return lax.bitwise_xor(ks, lax.bitwise_and(m, jnp.int32(0x7FFFFFFF)))


def _digit(ks, shift, first):
    b = lax.bitwise_and(lax.shift_right_logical(ks, shift), jnp.int32(15))
    if first:
        b = lax.bitwise_xor(b, jnp.int32(8))
    return b


def _popcnt(m):
    return plsc.all_reduce_population_count(m)


def _popcnt_s(m):
    return plsc.all_reduce_population_count(m, reduce=16).reshape(())


def _sc_body(dists_hbm, cb_hbm, tc_hbm, cw_hbm,
             d0, d1, hist, ck0, ci0, ck1, ci1, oidx, gidx, gidx2,
             grow, grow2, growc, sem0, sem1, gsem, gsem2):
    lane = lax.iota(jnp.int32, 16)
    ones16 = jnp.ones((16,), jnp.int32)
    zeros16 = jnp.zeros((16,), jnp.int32)
    wid = lax.axis_index("s") * _SC_CORES + lax.axis_index("c")
    base = wid * _ROWS_PER_W

    def scan_hist(k):
        hv = hist[pl.ds(0, 16)]
        cum = plsc.cumsum(hv)
        m = cum >= k
        bstar = jnp.max(plsc.all_reduce_ffs(m))
        cless = jnp.max(jnp.where(m, 0, cum))
        return bstar, cless

    def refine_level(shift, first, last, srck, srci, dstk, dsti,
                     offw, k_rem, ncand):
        hist[pl.ds(0, 16)] = zeros16
        niter = (ncand + 15) // 16

        def h(j, _):
            ks = srck[pl.ds(16 * j, 16)]
            valid = (lane + 16 * j) < ncand
            plsc.addupdate_scatter(hist, [_digit(ks, shift, first)], ones16,
                                   mask=valid)
            return 0
        lax.fori_loop(0, niter, h, 0)
        bstar, cless = scan_hist(k_rem)
        krem2 = zeros16 + (k_rem - cless)

        def e(j, carry):
            offw, offc, krem2 = carry
            ks = srck[pl.ds(16 * j, 16)]
            iv = srci[pl.ds(16 * j, 16)]
            valid = (lane + 16 * j) < ncand
            d = _digit(ks, shift, first)
            mw = jnp.logical_and(valid, d < bstar)
            mc = jnp.logical_and(valid, d == bstar)
            if last:
                trank = plsc.cumsum(jnp.where(mc, 1, 0))
                mtake = jnp.logical_and(mc, trank <= krem2)
                krem2 = krem2 - _popcnt(mtake)
                mw = jnp.logical_or(mw, mtake)
            posw = offw + plsc.cumsum(jnp.where(mw, 1, 0)) - 1
            plsc.store_scatter(oidx, [posw], iv, mask=mw)
            offw = offw + _popcnt(mw)
            if not last:
                posc = offc + plsc.cumsum(jnp.where(mc, 1, 0)) - 1
                plsc.store_scatter(dstk, [posc], ks, mask=mc)
                plsc.store_scatter(dsti, [posc], iv, mask=mc)
                offc = offc + _popcnt(mc)
            return offw, offc, krem2
        offw, offc, _ = lax.fori_loop(0, niter, e, (offw, zeros16, krem2))
        return offw, k_rem - cless, jnp.max(offc)

    def select_row(drow):
        def ga(i, acc):
            a0, a1, a2, a3 = acc
            a0 = jnp.minimum(a0, drow[pl.ds(64 * i, 16)])
            a1 = jnp.minimum(a1, drow[pl.ds(64 * i + 16, 16)])
            a2 = jnp.minimum(a2, drow[pl.ds(64 * i + 32, 16)])
            a3 = jnp.minimum(a3, drow[pl.ds(64 * i + 48, 16)])
            return a0, a1, a2, a3
        inf16 = jnp.full((16,), jnp.inf, jnp.float32)
        a0, a1, a2, a3 = lax.fori_loop(0, 128, ga, (inf16, inf16, inf16, inf16))
        t = jnp.max(jnp.maximum(jnp.maximum(a0, a1), jnp.maximum(a2, a3)))

        @plsc.parallel_loop(0, KCB // 16, unroll=8, carry=zeros16)
        def offc(j, offc):
            v = drow[pl.ds(16 * j, 16)]
            m = v <= t
            ks = _key_s(v)
            iv = lane + 16 * j
            posc = offc + plsc.cumsum(jnp.where(m, 1, 0)) - 1
            plsc.store_scatter(ck0, [posc], ks, mask=m)
            plsc.store_scatter(ci0, [posc], iv, mask=m)
            return offc + _popcnt(m)
        ncand = jnp.max(offc)

        offw = zeros16
        k_rem = jnp.int32(NCODES)
        bufs = [(ck0, ci0, ck1, ci1), (ck1, ci1, ck0, ci0)]
        for lvl in range(8):
            sk, si, dk, di = bufs[lvl % 2]
            if lvl < 7:
                offw, k_rem, ncand = refine_level(28 - 4 * lvl, lvl == 0, False,
                                                  sk, si, dk, di,
                                                  offw, k_rem, ncand)
            else:
                refine_level(0, False, True, sk, si, dk, di, offw, k_rem, ncand)

    gidxs = (gidx, gidx2)
    grows = (grow, grow2)
    gsems = (gsem, gsem2)

    def writeback(r, p):
        pltpu.make_async_copy(cb_hbm.at[gidxs[p]], grows[p], gsems[p]).wait()
        for q in range(NCODES):
            for h in range(4):
                growc[q, pl.ds(16 * h, 16)] = grows[p][q, pl.ds(16 * h, 16)]
        pltpu.sync_copy(growc, cw_hbm.at[base + r])
        pltpu.sync_copy(gidxs[p], tc_hbm.at[base + r])

    def handle_row(r, drow, dsem, p):
        pltpu.make_async_copy(dists_hbm.at[base + r], drow, dsem).wait()
        select_row(drow)
        for q in range(4):
            gidxs[p][pl.ds(16 * q, 16)] = oidx[pl.ds(16 * q, 16)]
        pltpu.make_async_copy(cb_hbm.at[gidxs[p]], grows[p], gsems[p]).start()

        @pl.when(r + 2 < _ROWS_PER_W)
        def _():
            pltpu.make_async_copy(dists_hbm.at[base + r + 2], drow, dsem).start()

        @pl.when(r >= 1)
        def _():
            writeback(r - 1, 1 - p)

    pltpu.make_async_copy(dists_hbm.at[base], d0, sem0).start()
    pltpu.make_async_copy(dists_hbm.at[base + 1], d1, sem1).start()

    def row_pair(i, _):
        handle_row(2 * i, d0, sem0, 0)
        handle_row(2 * i + 1, d1, sem1, 1)
        return 0
    lax.fori_loop(0, _ROWS_PER_W // 2, row_pair, 0)
    writeback(jnp.int32(_ROWS_PER_W - 1), 1)


def _sc_topk_gather(dists, codebook):
    mesh = plsc.VectorSubcoreMesh(core_axis_name="c", subcore_axis_name="s",
                                  num_cores=_SC_CORES,
                                  num_subcores=_SC_SUBCORES)
    f = pl.kernel(
        _sc_body,
        out_type=[
            jax.ShapeDtypeStruct((B, NCODES), jnp.int32),
            jax.ShapeDtypeStruct((B, NCODES, D), jnp.float32),
        ],
        mesh=mesh,
        compiler_params=pltpu.CompilerParams(needs_layout_passes=False),
        scratch_types=[
            pltpu.VMEM((KCB,), jnp.float32),
            pltpu.VMEM((KCB,), jnp.float32),
            pltpu.VMEM((256,), jnp.int32),
            pltpu.VMEM((KCB + 16,), jnp.int32),
            pltpu.VMEM((KCB + 16,), jnp.int32),
            pltpu.VMEM((KCB + 16,), jnp.int32),
            pltpu.VMEM((KCB + 16,), jnp.int32),
            pltpu.VMEM((NCODES + 16,), jnp.int32),
            pltpu.VMEM((NCODES,), jnp.int32),
            pltpu.VMEM((NCODES,), jnp.int32),
            pltpu.VMEM((NCODES, 128), jnp.float32),
            pltpu.VMEM((NCODES, 128), jnp.float32),
            pltpu.VMEM((NCODES, D), jnp.float32),
            pltpu.SemaphoreType.DMA,
            pltpu.SemaphoreType.DMA,
            pltpu.SemaphoreType.DMA,
            pltpu.SemaphoreType.DMA,
        ],
    )
    cb_pad = jnp.pad(codebook, ((0, 0), (0, 128 - D)))
    return f(dists, cb_pad)


_C_ROWS = 8
_NC = FIN * NCODES


def _mlp_body(cw_ref, xh_ref, x_ref, tc_ref, codes_ref,
              wi_ref, bi_ref, wc_ref, bc_ref, w1_ref, b1_ref,
              w2_ref, b2_ref, wo_ref, bo_ref,
              xo_ref, co_ref):
    n = _C_ROWS * _NC
    cwf = cw_ref[...].reshape(n, D)
    xh = xh_ref[...]
    xbf = jnp.broadcast_to(
        xh.reshape(_C_ROWS * FIN, 1, D), (_C_ROWS * FIN, NCODES, D)
    ).reshape(n, D)

    h = jnp.dot(cwf, wi_ref[...], preferred_element_type=jnp.float32) + bi_ref[...][None, :]
    hc = jnp.concatenate([h, xbf], axis=-1)
    h = jnp.dot(hc, wc_ref[...], preferred_element_type=jnp.float32) + bc_ref[...][None, :]
    r = jnp.maximum(jnp.dot(h, w1_ref[...], preferred_element_type=jnp.float32)
                    + b1_ref[...][None, :], 0.0)
    h = h + (jnp.dot(r, w2_ref[...], preferred_element_type=jnp.float32)
             + b2_ref[...][None, :])
    out = jnp.dot(h, wo_ref[...], preferred_element_type=jnp.float32) + bo_ref[...][None, :]
    out = out + 1.0 * cwf
    cand = out + xbf

    cn = jnp.sum(cand * cand, axis=-1)
    xv = x_ref[...]
    cross_full = lax.dot_general(cand, xv, (((1,), (1,)), ((), ())),
                                 preferred_element_type=jnp.float32)
    cross3 = cross_full.reshape(_C_ROWS, _NC, _C_ROWS)
    bsel = (lax.broadcasted_iota(jnp.int32, (_C_ROWS, _NC, _C_ROWS), 2)
            == lax.broadcasted_iota(jnp.int32, (_C_ROWS, _NC, _C_ROWS), 0))
    cross = jnp.sum(jnp.where(bsel, cross3, 0.0), axis=-1)
    dist = cn.reshape(_C_ROWS, _NC) - 2.0 * cross

    dcur = dist
    cols = lax.broadcasted_iota(jnp.int32, (_C_ROWS, _NC), 1)
    idx_list = []
    for _ in range(FOUT):
        it = jnp.argmin(dcur, axis=-1).astype(jnp.int32)
        idx_list.append(it)
        dcur = jnp.where(cols == it[:, None], jnp.inf, dcur)
    idx = jnp.stack(idx_list, axis=-1)

    cand3 = cand.reshape(_C_ROWS, _NC, D)
    tcf = tc_ref[...].astype(jnp.float32)
    sel_rows = []
    oh_cols = lax.broadcasted_iota(jnp.int32, (FOUT, _NC), 1)
    for b in range(_C_ROWS):
        ohb = (oh_cols == idx[b][:, None]).astype(jnp.float32)
        augb = jnp.concatenate([cand3[b], tcf[b][:, None]], axis=1)
        sel_rows.append(jnp.dot(ohb, augb, preferred_element_type=jnp.float32,
                                precision=lax.Precision.HIGHEST))
    sel = jnp.stack(sel_rows, axis=0)
    xo_ref[...] = sel[:, :, :D]
    codes_g = (sel[:, :, D] + 0.5).astype(jnp.int32)
    fidx = lax.shift_right_logical(idx, 6)
    fidxb = jnp.broadcast_to(fidx[None], (4, _C_ROWS, FOUT))
    csel = jnp.take_along_axis(codes_ref[...], fidxb, axis=-1)
    co_ref[...] = jnp.concatenate([csel, codes_g[None]], axis=0)


def _stage_c(cw, xhat_BFD, x_BD, tc_flat, codes_MBF,
             W_in, b_in, W_cat, b_cat, W_r1, b_r1, W_r2, b_r2, W_out, b_out):
    grid = BBASE // _C_ROWS
    const = lambda *dims: pl.BlockSpec(dims, lambda i: (0,) * len(dims))
    return pl.pallas_call(
        _mlp_body,
        grid=(grid,),
        in_specs=[
            pl.BlockSpec((_C_ROWS, _NC, D), lambda i: (i, 0, 0)),
            pl.BlockSpec((_C_ROWS, FIN, D), lambda i: (i, 0, 0)),
            pl.BlockSpec((_C_ROWS, D), lambda i: (i, 0)),
            pl.BlockSpec((_C_ROWS, _NC), lambda i: (i, 0)),
            pl.BlockSpec((4, _C_ROWS, FOUT), lambda i: (0, i, 0)),
            const(D, DH), const(DH), const(DH + D, DH), const(DH),
            const(DH, DFF), const(DFF), const(DFF, DH), const(DH),
            const(DH, D), const(D),
        ],
        out_specs=[
            pl.BlockSpec((_C_ROWS, FOUT, D), lambda i: (i, 0, 0)),
            pl.BlockSpec((5, _C_ROWS, FOUT), lambda i: (0, i, 0)),
        ],
        out_shape=[
            jax.ShapeDtypeStruct((BBASE, FOUT, D), jnp.float32),
            jax.ShapeDtypeStruct((5, BBASE, FOUT), jnp.int32),
        ],
    )(cw, xhat_BFD, x_BD, tc_flat, codes_MBF,
      W_in, b_in, W_cat, b_cat, W_r1, b_r1, W_r2, b_r2, W_out, b_out)


def kernel(x_BD, xhat_BFD, codes_MBF, codebook, codebook_rq,
           W_in, b_in, W_cat, b_cat, W_r1, b_r1, W_r2, b_r2, W_out, b_out):
    cn = jnp.sum(codebook_rq * codebook_rq, axis=-1)
    dists = _stage_a(x_BD, xhat_BFD, codebook_rq, cn)
    tc64, cw = _sc_topk_gather(dists, codebook)
    tc_flat = tc64.reshape(BBASE, _NC)
    cw3 = cw.reshape(BBASE, _NC, D)
    xhat_next, codes_out = _stage_c(
        cw3, xhat_BFD, x_BD, tc_flat, codes_MBF,
        W_in, b_in, W_cat, b_cat, W_r1, b_r1, W_r2, b_r2, W_out, b_out)
    return (xhat_next, codes_out)


# speedup vs baseline: 15.0929x; 1.0364x over previous
import functools

import jax
import jax.numpy as jnp
from jax import lax
from jax.experimental import pallas as pl
from jax.experimental.pallas import tpu as pltpu
from jax.experimental.pallas import tpu_sc as plsc

D = 64
DH = 64
DFF = 256
KCB = 8192
NCODES = 64
BBASE = 256
FIN = 16
FOUT = 16
B = BBASE * FIN


_A_ROWS = 16


def _dists_body(x_ref, xh_ref, rq_ref, cn_ref, out_ref):
    xt = (x_ref[...][:, None, :] - xh_ref[...]).reshape(_A_ROWS * FIN, D)
    prod = lax.dot_general(xt, rq_ref[...], (((1,), (1,)), ((), ())),
                           preferred_element_type=jnp.float32)
    out_ref[...] = cn_ref[...][None, :] - 2.0 * prod


def _stage_a(x_BD, xhat_BFD, codebook_rq, cn):
    grid = BBASE // _A_ROWS
    return pl.pallas_call(
        _dists_body,
        grid=(grid,),
        in_specs=[
            pl.BlockSpec((_A_ROWS, D), lambda i: (i, 0)),
            pl.BlockSpec((_A_ROWS, FIN, D), lambda i: (i, 0, 0)),
            pl.BlockSpec((KCB, D), lambda i: (0, 0)),
            pl.BlockSpec((KCB,), lambda i: (0,)),
        ],
        out_specs=pl.BlockSpec((_A_ROWS * FIN, KCB), lambda i: (i, 0)),
        out_shape=jax.ShapeDtypeStruct((B, KCB), jnp.float32),
    )(x_BD, xhat_BFD, codebook_rq, cn)


_SC_CORES = 2
_SC_SUBCORES = 16
_SC_WORKERS = _SC_CORES * _SC_SUBCORES
_ROWS_PER_W = B // _SC_WORKERS


def _key_s(v):
    ks = plsc.bitcast(v, jnp.int32)
    m = lax.shift_right_arithmetic(ks, 31)
    return lax.bitwise_xor(ks, lax.bitwise_and(m, jnp.int32(0x7FFFFFFF)))


def _digit(ks, shift, first):
    b = lax.bitwise_and(lax.shift_right_logical(ks, shift), jnp.int32(15))
    if first:
        b = lax.bitwise_xor(b, jnp.int32(8))
    return b


def _popcnt(m):
    return plsc.all_reduce_population_count(m)


def _popcnt_s(m):
    return plsc.all_reduce_population_count(m, reduce=16).reshape(())


def _sc_body(dists_hbm, cb_hbm, tc_hbm, cw_hbm,
             d0, d1, hist, ck0, ci0, ck1, ci1, oidx, gidx, gidx2,
             grow, grow2, growc, sem0, sem1, gsem, gsem2):
    lane = lax.iota(jnp.int32, 16)
    ones16 = jnp.ones((16,), jnp.int32)
    zeros16 = jnp.zeros((16,), jnp.int32)
    wid = lax.axis_index("s") * _SC_CORES + lax.axis_index("c")
    base = wid * _ROWS_PER_W

    def scan_hist(k):
        hv = hist[pl.ds(0, 16)]
        cum = plsc.cumsum(hv)
        m = cum >= k
        bstar = jnp.max(plsc.all_reduce_ffs(m))
        cless = jnp.max(jnp.where(m, 0, cum))
        return bstar, cless

    def refine_level(shift, first, last, srck, srci, dstk, dsti,
                     offw, k_rem, ncand):
        hist[pl.ds(0, 16)] = zeros16
        niter = (ncand + 15) // 16

        def h(j, _):
            ks = srck[pl.ds(16 * j, 16)]
            valid = (lane + 16 * j) < ncand
            plsc.addupdate_scatter(hist, [_digit(ks, shift, first)], ones16,
                                   mask=valid)
            return 0
        lax.fori_loop(0, niter, h, 0)
        bstar, cless = scan_hist(k_rem)
        krem2 = zeros16 + (k_rem - cless)

        def e(j, carry):
            offw, offc, krem2 = carry
            ks = srck[pl.ds(16 * j, 16)]
            iv = srci[pl.ds(16 * j, 16)]
            valid = (lane + 16 * j) < ncand
            d = _digit(ks, shift, first)
            mw = jnp.logical_and(valid, d < bstar)
            mc = jnp.logical_and(valid, d == bstar)
            if last:
                trank = plsc.cumsum(jnp.where(mc, 1, 0))
                mtake = jnp.logical_and(mc, trank <= krem2)
                krem2 = krem2 - _popcnt(mtake)
                mw = jnp.logical_or(mw, mtake)
            posw = offw + plsc.cumsum(jnp.where(mw, 1, 0)) - 1
            plsc.store_scatter(oidx, [posw], iv, mask=mw)
            offw = offw + _popcnt(mw)
            if not last:
                posc = offc + plsc.cumsum(jnp.where(mc, 1, 0)) - 1
                plsc.store_scatter(dstk, [posc], ks, mask=mc)
                plsc.store_scatter(dsti, [posc], iv, mask=mc)
                offc = offc + _popcnt(mc)
            return offw, offc, krem2
        offw, offc, _ = lax.fori_loop(0, niter, e, (offw, zeros16, krem2))
        return offw, k_rem - cless, jnp.max(offc)

    def select_row(drow):
        inf16 = jnp.full((16,), jnp.inf, jnp.float32)

        @plsc.parallel_loop(0, 128, unroll=8, carry=(inf16, inf16, inf16, inf16))
        def ga_acc(i, acc):
            a0, a1, a2, a3 = acc
            a0 = jnp.minimum(a0, drow[pl.ds(64 * i, 16)])
            a1 = jnp.minimum(a1, drow[pl.ds(64 * i + 16, 16)])
            a2 = jnp.minimum(a2, drow[pl.ds(64 * i + 32, 16)])
            a3 = jnp.minimum(a3, drow[pl.ds(64 * i + 48, 16)])
            return a0, a1, a2, a3
        a0, a1, a2, a3 = ga_acc
        t = jnp.max(jnp.maximum(jnp.maximum(a0, a1), jnp.maximum(a2, a3)))

        @plsc.parallel_loop(0, KCB // 16, unroll=8, carry=zeros16)
        def offc(j, offc):
            v = drow[pl.ds(16 * j, 16)]
            m = v <= t
            ks = _key_s(v)
            iv = lane + 16 * j
            posc = offc + plsc.cumsum(jnp.where(m, 1, 0)) - 1
            plsc.store_scatter(ck0, [posc], ks, mask=m)
            plsc.store_scatter(ci0, [posc], iv, mask=m)
            return offc + _popcnt(m)
        ncand = jnp.max(offc)

        offw = zeros16
        k_rem = jnp.int32(NCODES)
        bufs = [(ck0, ci0, ck1, ci1), (ck1, ci1, ck0, ci0)]
        for lvl in range(8):
            sk, si, dk, di = bufs[lvl % 2]
            if lvl < 7:
                offw, k_rem, ncand = refine_level(28 - 4 * lvl, lvl == 0, False,
                                                  sk, si, dk, di,
                                                  offw, k_rem, ncand)
            else:
                refine_level(0, False, True, sk, si, dk, di, offw, k_rem, ncand)

    gidxs = (gidx, gidx2)
    grows = (grow, grow2)
    gsems = (gsem, gsem2)

    def writeback(r, p):
        pltpu.make_async_copy(cb_hbm.at[gidxs[p]], grows[p], gsems[p]).wait()

        @plsc.parallel_loop(0, NCODES, unroll=4, carry=None)
        def _(q):
            for h in range(4):
                growc[q, pl.ds(16 * h, 16)] = grows[p][q, pl.ds(16 * h, 16)]
        pltpu.sync_copy(growc, cw_hbm.at[base + r])
        pltpu.sync_copy(gidxs[p], tc_hbm.at[base + r])

    def handle_row(r, drow, dsem, p):
        pltpu.make_async_copy(dists_hbm.at[base + r], drow, dsem).wait()
        select_row(drow)
        for q in range(4):
            gidxs[p][pl.ds(16 * q, 16)] = oidx[pl.ds(16 * q, 16)]
        pltpu.make_async_copy(cb_hbm.at[gidxs[p]], grows[p], gsems[p]).start()

        @pl.when(r + 2 < _ROWS_PER_W)
        def _():
            pltpu.make_async_copy(dists_hbm.at[base + r + 2], drow, dsem).start()

        @pl.when(r >= 1)
        def _():
            writeback(r - 1, 1 - p)

    pltpu.make_async_copy(dists_hbm.at[base], d0, sem0).start()
    pltpu.make_async_copy(dists_hbm.at[base + 1], d1, sem1).start()

    def row_pair(i, _):
        handle_row(2 * i, d0, sem0, 0)
        handle_row(2 * i + 1, d1, sem1, 1)
        return 0
    lax.fori_loop(0, _ROWS_PER_W // 2, row_pair, 0)
    writeback(jnp.int32(_ROWS_PER_W - 1), 1)


def _sc_topk_gather(dists, codebook):
    mesh = plsc.VectorSubcoreMesh(core_axis_name="c", subcore_axis_name="s",
                                  num_cores=_SC_CORES,
                                  num_subcores=_SC_SUBCORES)
    f = pl.kernel(
        _sc_body,
        out_type=[
            jax.ShapeDtypeStruct((B, NCODES), jnp.int32),
            jax.ShapeDtypeStruct((B, NCODES, D), jnp.float32),
        ],
        mesh=mesh,
        compiler_params=pltpu.CompilerParams(needs_layout_passes=False),
        scratch_types=[
            pltpu.VMEM((KCB,), jnp.float32),
            pltpu.VMEM((KCB,), jnp.float32),
            pltpu.VMEM((256,), jnp.int32),
            pltpu.VMEM((KCB + 16,), jnp.int32),
            pltpu.VMEM((KCB + 16,), jnp.int32),
            pltpu.VMEM((KCB + 16,), jnp.int32),
            pltpu.VMEM((KCB + 16,), jnp.int32),
            pltpu.VMEM((NCODES + 16,), jnp.int32),
            pltpu.VMEM((NCODES,), jnp.int32),
            pltpu.VMEM((NCODES,), jnp.int32),
            pltpu.VMEM((NCODES, 128), jnp.float32),
            pltpu.VMEM((NCODES, 128), jnp.float32),
            pltpu.VMEM((NCODES, D), jnp.float32),
            pltpu.SemaphoreType.DMA,
            pltpu.SemaphoreType.DMA,
            pltpu.SemaphoreType.DMA,
            pltpu.SemaphoreType.DMA,
        ],
    )
    cb_pad = jnp.pad(codebook, ((0, 0), (0, 128 - D)))
    return f(dists, cb_pad)


_C_ROWS = 8
_NC = FIN * NCODES


def _mlp_body(cw_ref, xh_ref, x_ref, tc_ref, codes_ref,
              wi_ref, bi_ref, wc_ref, bc_ref, w1_ref, b1_ref,
              w2_ref, b2_ref, wo_ref, bo_ref,
              xo_ref, co_ref):
    n = _C_ROWS * _NC
    cwf = cw_ref[...].reshape(n, D)
    xh = xh_ref[...]
    xbf = jnp.broadcast_to(
        xh.reshape(_C_ROWS * FIN, 1, D), (_C_ROWS * FIN, NCODES, D)
    ).reshape(n, D)

    h = jnp.dot(cwf, wi_ref[...], preferred_element_type=jnp.float32) + bi_ref[...][None, :]
    hc = jnp.concatenate([h, xbf], axis=-1)
    h = jnp.dot(hc, wc_ref[...], preferred_element_type=jnp.float32) + bc_ref[...][None, :]
    r = jnp.maximum(jnp.dot(h, w1_ref[...], preferred_element_type=jnp.float32)
                    + b1_ref[...][None, :], 0.0)
    h = h + (jnp.dot(r, w2_ref[...], preferred_element_type=jnp.float32)
             + b2_ref[...][None, :])
    out = jnp.dot(h, wo_ref[...], preferred_element_type=jnp.float32) + bo_ref[...][None, :]
    out = out + 1.0 * cwf
    cand = out + xbf

    cn = jnp.sum(cand * cand, axis=-1)
    xv = x_ref[...]
    cross_full = lax.dot_general(cand, xv, (((1,), (1,)), ((), ())),
                                 preferred_element_type=jnp.float32)
    cross3 = cross_full.reshape(_C_ROWS, _NC, _C_ROWS)
    bsel = (lax.broadcasted_iota(jnp.int32, (_C_ROWS, _NC, _C_ROWS), 2)
            == lax.broadcasted_iota(jnp.int32, (_C_ROWS, _NC, _C_ROWS), 0))
    cross = jnp.sum(jnp.where(bsel, cross3, 0.0), axis=-1)
    dist = cn.reshape(_C_ROWS, _NC) - 2.0 * cross

    dcur = dist
    cols = lax.broadcasted_iota(jnp.int32, (_C_ROWS, _NC), 1)
    idx_list = []
    for _ in range(FOUT):
        it = jnp.argmin(dcur, axis=-1).astype(jnp.int32)
        idx_list.append(it)
        dcur = jnp.where(cols == it[:, None], jnp.inf, dcur)
    idx = jnp.stack(idx_list, axis=-1)

    cand3 = cand.reshape(_C_ROWS, _NC, D)
    tcf = tc_ref[...].astype(jnp.float32)
    sel_rows = []
    oh_cols = lax.broadcasted_iota(jnp.int32, (FOUT, _NC), 1)
    for b in range(_C_ROWS):
        ohb = (oh_cols == idx[b][:, None]).astype(jnp.float32)
        augb = jnp.concatenate([cand3[b], tcf[b][:, None]], axis=1)
        sel_rows.append(jnp.dot(ohb, augb, preferred_element_type=jnp.float32,
                                precision=lax.Precision.HIGHEST))
    sel = jnp.stack(sel_rows, axis=0)
    xo_ref[...] = sel[:, :, :D]
    codes_g = (sel[:, :, D] + 0.5).astype(jnp.int32)
    fidx = lax.shift_right_logical(idx, 6)
    fidxb = jnp.broadcast_to(fidx[None], (4, _C_ROWS, FOUT))
    csel = jnp.take_along_axis(codes_ref[...], fidxb, axis=-1)
    co_ref[...] = jnp.concatenate([csel, codes_g[None]], axis=0)


def _stage_c(cw, xhat_BFD, x_BD, tc_flat, codes_MBF,
             W_in, b_in, W_cat, b_cat, W_r1, b_r1, W_r2, b_r2, W_out, b_out):
    grid = BBASE // _C_ROWS
    const = lambda *dims: pl.BlockSpec(dims, lambda i: (0,) * len(dims))
    return pl.pallas_call(
        _mlp_body,
        grid=(grid,),
        in_specs=[
            pl.BlockSpec((_C_ROWS, _NC, D), lambda i: (i, 0, 0)),
            pl.BlockSpec((_C_ROWS, FIN, D), lambda i: (i, 0, 0)),
            pl.BlockSpec((_C_ROWS, D), lambda i: (i, 0)),
            pl.BlockSpec((_C_ROWS, _NC), lambda i: (i, 0)),
            pl.BlockSpec((4, _C_ROWS, FOUT), lambda i: (0, i, 0)),
            const(D, DH), const(DH), const(DH + D, DH), const(DH),
            const(DH, DFF), const(DFF), const(DFF, DH), const(DH),
            const(DH, D), const(D),
        ],
        out_specs=[
            pl.BlockSpec((_C_ROWS, FOUT, D), lambda i: (i, 0, 0)),
            pl.BlockSpec((5, _C_ROWS, FOUT), lambda i: (0, i, 0)),
        ],
        out_shape=[
            jax.ShapeDtypeStruct((BBASE, FOUT, D), jnp.float32),
            jax.ShapeDtypeStruct((5, BBASE, FOUT), jnp.int32),
        ],
    )(cw, xhat_BFD, x_BD, tc_flat, codes_MBF,
      W_in, b_in, W_cat, b_cat, W_r1, b_r1, W_r2, b_r2, W_out, b_out)


def kernel(x_BD, xhat_BFD, codes_MBF, codebook, codebook_rq,
           W_in, b_in, W_cat, b_cat, W_r1, b_r1, W_r2, b_r2, W_out, b_out):
    cn = jnp.sum(codebook_rq * codebook_rq, axis=-1)
    dists = _stage_a(x_BD, xhat_BFD, codebook_rq, cn)
    tc64, cw = _sc_topk_gather(dists, codebook)
    tc_flat = tc64.reshape(BBASE, _NC)
    cw3 = cw.reshape(BBASE, _NC, D)
    xhat_next, codes_out = _stage_c(
        cw3, xhat_BFD, x_BD, tc_flat, codes_MBF,
        W_in, b_in, W_cat, b_cat, W_r1, b_r1, W_r2, b_r2, W_out, b_out)
    return (xhat_next, codes_out)


# speedup vs baseline: 15.9640x; 1.0577x over previous
import functools

import jax
import jax.numpy as jnp
from jax import lax
from jax.experimental import pallas as pl
from jax.experimental.pallas import tpu as pltpu
from jax.experimental.pallas import tpu_sc as plsc

D = 64
DH = 64
DFF = 256
KCB = 8192
NCODES = 64
BBASE = 256
FIN = 16
FOUT = 16
B = BBASE * FIN


_A_ROWS = 16


def _dists_body(x_ref, xh_ref, rq_ref, cn_ref, out_ref):
    xt = (x_ref[...][:, None, :] - xh_ref[...]).reshape(_A_ROWS * FIN, D)
    prod = lax.dot_general(xt, rq_ref[...], (((1,), (1,)), ((), ())),
                           preferred_element_type=jnp.float32)
    out_ref[...] = cn_ref[...][None, :] - 2.0 * prod


def _stage_a(x_BD, xhat_BFD, codebook_rq, cn):
    grid = BBASE // _A_ROWS
    return pl.pallas_call(
        _dists_body,
        grid=(grid,),
        in_specs=[
            pl.BlockSpec((_A_ROWS, D), lambda i: (i, 0)),
            pl.BlockSpec((_A_ROWS, FIN, D), lambda i: (i, 0, 0)),
            pl.BlockSpec((KCB, D), lambda i: (0, 0)),
            pl.BlockSpec((KCB,), lambda i: (0,)),
        ],
        out_specs=pl.BlockSpec((_A_ROWS * FIN, KCB), lambda i: (i, 0)),
        out_shape=jax.ShapeDtypeStruct((B, KCB), jnp.float32),
    )(x_BD, xhat_BFD, codebook_rq, cn)


_SC_CORES = 2
_SC_SUBCORES = 16
_SC_WORKERS = _SC_CORES * _SC_SUBCORES
_ROWS_PER_W = B // _SC_WORKERS


def _key_s(v):
    ks = plsc.bitcast(v, jnp.int32)
    m = lax.shift_right_arithmetic(ks, 31)
    return lax.bitwise_xor(ks, lax.bitwise_and(m, jnp.int32(0x7FFFFFFF)))


def _digit(ks, shift, first):
    b = lax.bitwise_and(lax.shift_right_logical(ks, shift), jnp.int32(15))
    if first:
        b = lax.bitwise_xor(b, jnp.int32(8))
    return b


def _popcnt(m):
    return plsc.all_reduce_population_count(m)


def _popcnt_s(m):
    return plsc.all_reduce_population_count(m, reduce=16).reshape(())


def _sc_body(dists_hbm, cb_hbm, tc_hbm, cw_hbm,
             d0, d1, hist, ck0, ci0, ck1, ci1, oidx, gidx, gidx2,
             grow, grow2, growc, sem0, sem1, gsem, gsem2):
    lane = lax.iota(jnp.int32, 16)
    ones16 = jnp.ones((16,), jnp.int32)
    zeros16 = jnp.zeros((16,), jnp.int32)
    wid = lax.axis_index("s") * _SC_CORES + lax.axis_index("c")
    base = wid * _ROWS_PER_W

    def scan_hist(k):
        hv = hist[pl.ds(0, 16)]
        cum = plsc.cumsum(hv)
        m = cum >= k
        bstar = jnp.max(plsc.all_reduce_ffs(m))
        cless = jnp.max(jnp.where(m, 0, cum))
        return bstar, cless

    def refine_level(shift, first, last, srck, srci, dstk, dsti,
                     offw, k_rem, ncand):
        hist[pl.ds(0, 16)] = zeros16
        niter = (ncand + 15) // 16

        def h(j, _):
            ks = srck[pl.ds(16 * j, 16)]
            valid = (lane + 16 * j) < ncand
            plsc.addupdate_scatter(hist, [_digit(ks, shift, first)], ones16,
                                   mask=valid)
            return 0
        lax.fori_loop(0, niter, h, 0)
        bstar, cless = scan_hist(k_rem)
        krem2 = zeros16 + (k_rem - cless)

        def e(j, carry):
            offw, offc, krem2 = carry
            ks = srck[pl.ds(16 * j, 16)]
            iv = srci[pl.ds(16 * j, 16)]
            valid = (lane + 16 * j) < ncand
            d = _digit(ks, shift, first)
            mw = jnp.logical_and(valid, d < bstar)
            mc = jnp.logical_and(valid, d == bstar)
            if last:
                trank = plsc.cumsum(jnp.where(mc, 1, 0))
                mtake = jnp.logical_and(mc, trank <= krem2)
                krem2 = krem2 - _popcnt(mtake)
                mw = jnp.logical_or(mw, mtake)
            posw = offw + plsc.cumsum(jnp.where(mw, 1, 0)) - 1
            plsc.store_scatter(oidx, [posw], iv, mask=mw)
            offw = offw + _popcnt(mw)
            if not last:
                posc = offc + plsc.cumsum(jnp.where(mc, 1, 0)) - 1
                plsc.store_scatter(dstk, [posc], ks, mask=mc)
                plsc.store_scatter(dsti, [posc], iv, mask=mc)
                offc = offc + _popcnt(mc)
            return offw, offc, krem2
        if last:
            offw, offc, _ = lax.fori_loop(0, niter, e, (offw, zeros16, krem2))
        else:
            @plsc.parallel_loop(0, niter, unroll=2, carry=(offw, zeros16, krem2))
            def eout(j, carry):
                return e(j, carry)
            offw, offc, _ = eout
        return offw, k_rem - cless, jnp.max(offc)

    def select_row(drow):
        inf16 = jnp.full((16,), jnp.inf, jnp.float32)

        @plsc.parallel_loop(0, 128, unroll=8, carry=(inf16, inf16, inf16, inf16))
        def ga_acc(i, acc):
            a0, a1, a2, a3 = acc
            a0 = jnp.minimum(a0, drow[pl.ds(64 * i, 16)])
            a1 = jnp.minimum(a1, drow[pl.ds(64 * i + 16, 16)])
            a2 = jnp.minimum(a2, drow[pl.ds(64 * i + 32, 16)])
            a3 = jnp.minimum(a3, drow[pl.ds(64 * i + 48, 16)])
            return a0, a1, a2, a3
        a0, a1, a2, a3 = ga_acc
        t = jnp.max(jnp.maximum(jnp.maximum(a0, a1), jnp.maximum(a2, a3)))

        @plsc.parallel_loop(0, KCB // 16, unroll=8, carry=zeros16)
        def offc(j, offc):
            v = drow[pl.ds(16 * j, 16)]
            m = v <= t
            ks = _key_s(v)
            iv = lane + 16 * j
            posc = offc + plsc.cumsum(jnp.where(m, 1, 0)) - 1
            plsc.store_scatter(ck0, [posc], ks, mask=m)
            plsc.store_scatter(ci0, [posc], iv, mask=m)
            return offc + _popcnt(m)
        ncand = jnp.max(offc)

        offw = zeros16
        k_rem = jnp.int32(NCODES)
        bufs = [(ck0, ci0, ck1, ci1), (ck1, ci1, ck0, ci0)]
        for lvl in range(8):
            sk, si, dk, di = bufs[lvl % 2]
            if lvl < 7:
                offw, k_rem, ncand = refine_level(28 - 4 * lvl, lvl == 0, False,
                                                  sk, si, dk, di,
                                                  offw, k_rem, ncand)
            else:
                refine_level(0, False, True, sk, si, dk, di, offw, k_rem, ncand)

    gidxs = (gidx, gidx2)
    grows = (grow, grow2)
    gsems = (gsem, gsem2)

    def writeback(r, p):
        pltpu.make_async_copy(cb_hbm.at[gidxs[p]], grows[p], gsems[p]).wait()

        @plsc.parallel_loop(0, NCODES, unroll=4, carry=None)
        def _(q):
            for h in range(4):
                growc[q, pl.ds(16 * h, 16)] = grows[p][q, pl.ds(16 * h, 16)]
        pltpu.sync_copy(growc, cw_hbm.at[base + r])
        pltpu.sync_copy(gidxs[p], tc_hbm.at[base + r])

    def handle_row(r, drow, dsem, p):
        pltpu.make_async_copy(dists_hbm.at[base + r], drow, dsem).wait()
        select_row(drow)
        for q in range(4):
            gidxs[p][pl.ds(16 * q, 16)] = oidx[pl.ds(16 * q, 16)]
        pltpu.make_async_copy(cb_hbm.at[gidxs[p]], grows[p], gsems[p]).start()

        @pl.when(r + 2 < _ROWS_PER_W)
        def _():
            pltpu.make_async_copy(dists_hbm.at[base + r + 2], drow, dsem).start()

        @pl.when(r >= 1)
        def _():
            writeback(r - 1, 1 - p)

    pltpu.make_async_copy(dists_hbm.at[base], d0, sem0).start()
    pltpu.make_async_copy(dists_hbm.at[base + 1], d1, sem1).start()

    def row_pair(i, _):
        handle_row(2 * i, d0, sem0, 0)
        handle_row(2 * i + 1, d1, sem1, 1)
        return 0
    lax.fori_loop(0, _ROWS_PER_W // 2, row_pair, 0)
    writeback(jnp.int32(_ROWS_PER_W - 1), 1)


def _sc_topk_gather(dists, codebook):
    mesh = plsc.VectorSubcoreMesh(core_axis_name="c", subcore_axis_name="s",
                                  num_cores=_SC_CORES,
                                  num_subcores=_SC_SUBCORES)
    f = pl.kernel(
        _sc_body,
        out_type=[
            jax.ShapeDtypeStruct((B, NCODES), jnp.int32),
            jax.ShapeDtypeStruct((B, NCODES, D), jnp.float32),
        ],
        mesh=mesh,
        compiler_params=pltpu.CompilerParams(needs_layout_passes=False),
        scratch_types=[
            pltpu.VMEM((KCB,), jnp.float32),
            pltpu.VMEM((KCB,), jnp.float32),
            pltpu.VMEM((256,), jnp.int32),
            pltpu.VMEM((KCB + 16,), jnp.int32),
            pltpu.VMEM((KCB + 16,), jnp.int32),
            pltpu.VMEM((KCB + 16,), jnp.int32),
            pltpu.VMEM((KCB + 16,), jnp.int32),
            pltpu.VMEM((NCODES + 16,), jnp.int32),
            pltpu.VMEM((NCODES,), jnp.int32),
            pltpu.VMEM((NCODES,), jnp.int32),
            pltpu.VMEM((NCODES, 128), jnp.float32),
            pltpu.VMEM((NCODES, 128), jnp.float32),
            pltpu.VMEM((NCODES, D), jnp.float32),
            pltpu.SemaphoreType.DMA,
            pltpu.SemaphoreType.DMA,
            pltpu.SemaphoreType.DMA,
            pltpu.SemaphoreType.DMA,
        ],
    )
    cb_pad = jnp.pad(codebook, ((0, 0), (0, 128 - D)))
    return f(dists, cb_pad)


_C_ROWS = 8
_NC = FIN * NCODES


def _mlp_body(cw_ref, xh_ref, x_ref, tc_ref, codes_ref,
              wi_ref, bi_ref, wc_ref, bc_ref, w1_ref, b1_ref,
              w2_ref, b2_ref, wo_ref, bo_ref,
              xo_ref, co_ref):
    n = _C_ROWS * _NC
    cwf = cw_ref[...].reshape(n, D)
    xh = xh_ref[...]
    xbf = jnp.broadcast_to(
        xh.reshape(_C_ROWS * FIN, 1, D), (_C_ROWS * FIN, NCODES, D)
    ).reshape(n, D)

    h = jnp.dot(cwf, wi_ref[...], preferred_element_type=jnp.float32) + bi_ref[...][None, :]
    hc = jnp.concatenate([h, xbf], axis=-1)
    h = jnp.dot(hc, wc_ref[...], preferred_element_type=jnp.float32) + bc_ref[...][None, :]
    r = jnp.maximum(jnp.dot(h, w1_ref[...], preferred_element_type=jnp.float32)
                    + b1_ref[...][None, :], 0.0)
    h = h + (jnp.dot(r, w2_ref[...], preferred_element_type=jnp.float32)
             + b2_ref[...][None, :])
    out = jnp.dot(h, wo_ref[...], preferred_element_type=jnp.float32) + bo_ref[...][None, :]
    out = out + 1.0 * cwf
    cand = out + xbf

    cn = jnp.sum(cand * cand, axis=-1)
    xv = x_ref[...]
    cross_full = lax.dot_general(cand, xv, (((1,), (1,)), ((), ())),
                                 preferred_element_type=jnp.float32)
    cross3 = cross_full.reshape(_C_ROWS, _NC, _C_ROWS)
    bsel = (lax.broadcasted_iota(jnp.int32, (_C_ROWS, _NC, _C_ROWS), 2)
            == lax.broadcasted_iota(jnp.int32, (_C_ROWS, _NC, _C_ROWS), 0))
    cross = jnp.sum(jnp.where(bsel, cross3, 0.0), axis=-1)
    dist = cn.reshape(_C_ROWS, _NC) - 2.0 * cross

    dcur = dist
    cols = lax.broadcasted_iota(jnp.int32, (_C_ROWS, _NC), 1)
    idx_list = []
    for _ in range(FOUT):
        it = jnp.argmin(dcur, axis=-1).astype(jnp.int32)
        idx_list.append(it)
        dcur = jnp.where(cols == it[:, None], jnp.inf, dcur)
    idx = jnp.stack(idx_list, axis=-1)

    cand3 = cand.reshape(_C_ROWS, _NC, D)
    tcv = tc_ref[...]
    sel_rows = []
    cg_rows = []
    oh_cols = lax.broadcasted_iota(jnp.int32, (FOUT, _NC), 1)
    for b in range(_C_ROWS):
        ohm = oh_cols == idx[b][:, None]
        sel_rows.append(jnp.dot(ohm.astype(jnp.float32), cand3[b],
                                preferred_element_type=jnp.float32))
        cg_rows.append(jnp.sum(jnp.where(ohm, tcv[b][None, :], 0), axis=-1))
    xo_ref[...] = jnp.stack(sel_rows, axis=0)
    codes_g = jnp.stack(cg_rows, axis=0)
    fidx = lax.shift_right_logical(idx, 6)
    fidxb = jnp.broadcast_to(fidx[None], (4, _C_ROWS, FOUT))
    csel = jnp.take_along_axis(codes_ref[...], fidxb, axis=-1)
    co_ref[...] = jnp.concatenate([csel, codes_g[None]], axis=0)


def _stage_c(cw, xhat_BFD, x_BD, tc_flat, codes_MBF,
             W_in, b_in, W_cat, b_cat, W_r1, b_r1, W_r2, b_r2, W_out, b_out):
    grid = BBASE // _C_ROWS
    const = lambda *dims: pl.BlockSpec(dims, lambda i: (0,) * len(dims))
    return pl.pallas_call(
        _mlp_body,
        grid=(grid,),
        in_specs=[
            pl.BlockSpec((_C_ROWS, _NC, D), lambda i: (i, 0, 0)),
            pl.BlockSpec((_C_ROWS, FIN, D), lambda i: (i, 0, 0)),
            pl.BlockSpec((_C_ROWS, D), lambda i: (i, 0)),
            pl.BlockSpec((_C_ROWS, _NC), lambda i: (i, 0)),
            pl.BlockSpec((4, _C_ROWS, FOUT), lambda i: (0, i, 0)),
            const(D, DH), const(DH), const(DH + D, DH), const(DH),
            const(DH, DFF), const(DFF), const(DFF, DH), const(DH),
            const(DH, D), const(D),
        ],
        out_specs=[
            pl.BlockSpec((_C_ROWS, FOUT, D), lambda i: (i, 0, 0)),
            pl.BlockSpec((5, _C_ROWS, FOUT), lambda i: (0, i, 0)),
        ],
        out_shape=[
            jax.ShapeDtypeStruct((BBASE, FOUT, D), jnp.float32),
            jax.ShapeDtypeStruct((5, BBASE, FOUT), jnp.int32),
        ],
    )(cw, xhat_BFD, x_BD, tc_flat, codes_MBF,
      W_in, b_in, W_cat, b_cat, W_r1, b_r1, W_r2, b_r2, W_out, b_out)


def kernel(x_BD, xhat_BFD, codes_MBF, codebook, codebook_rq,
           W_in, b_in, W_cat, b_cat, W_r1, b_r1, W_r2, b_r2, W_out, b_out):
    cn = jnp.sum(codebook_rq * codebook_rq, axis=-1)
    dists = _stage_a(x_BD, xhat_BFD, codebook_rq, cn)
    tc64, cw = _sc_topk_gather(dists, codebook)
    tc_flat = tc64.reshape(BBASE, _NC)
    cw3 = cw.reshape(BBASE, _NC, D)
    xhat_next, codes_out = _stage_c(
        cw3, xhat_BFD, x_BD, tc_flat, codes_MBF,
        W_in, b_in, W_cat, b_cat, W_r1, b_r1, W_r2, b_r2, W_out, b_out)
    return (xhat_next, codes_out)
